# baseline (speedup 1.0000x reference)
import functools
import jax
import jax.numpy as jnp
from jax.experimental import pallas as pl
from jax.experimental.pallas import tpu as pltpu

_NEG = 0.2


def _matmul_kernel(x_ref, w_ref, o_ref, acc_ref, *, nsteps):
    k = pl.program_id(2)

    @pl.when(k == 0)
    def _():
        acc_ref[...] = jnp.zeros_like(acc_ref)

    acc_ref[...] += jnp.dot(x_ref[...], w_ref[...],
                            preferred_element_type=jnp.float32)

    @pl.when(k == nsteps - 1)
    def _():
        o_ref[...] = acc_ref[...]


def _matmul(x, w, bm=256, bn=512, bk=128):
    m, k = x.shape
    k2, n = w.shape
    assert k == k2
    nsteps = pl.cdiv(k, bk)
    return pl.pallas_call(
        functools.partial(_matmul_kernel, nsteps=nsteps),
        grid=(pl.cdiv(m, bm), pl.cdiv(n, bn), nsteps),
        in_specs=[
            pl.BlockSpec((bm, bk), lambda i, j, kk: (i, kk)),
            pl.BlockSpec((bk, bn), lambda i, j, kk: (kk, j)),
        ],
        out_specs=pl.BlockSpec((bm, bn), lambda i, j, kk: (i, j)),
        scratch_shapes=[pltpu.VMEM((bm, bn), jnp.float32)],
        out_shape=jax.ShapeDtypeStruct((m, n), jnp.float32),
        compiler_params=pltpu.CompilerParams(
            dimension_semantics=("parallel", "parallel", "arbitrary")),
    )(x, w)


def _gat_layer(x, src, dst, edge_attr, W, We, a_src, a_dst, a_e, b, H, C):
    n = x.shape[0]
    d = x.shape[1]
    w_s = (W.reshape(d, H, C) * a_src[None]).sum(-1)
    w_d = (W.reshape(d, H, C) * a_dst[None]).sum(-1)
    pad = 128 - 2 * H
    w_aug = jnp.concatenate(
        [W, w_s, w_d, jnp.zeros((d, pad), jnp.float32)], axis=1)
    hh = _matmul(x, w_aug)
    h = hh[:, :H * C]
    s_src = hh[:, H * C:H * C + H]
    s_dst = hh[:, H * C + H:H * C + 2 * H]

    we_a = (We.reshape(-1, H, C) * a_e[None]).sum(-1)
    eal = edge_attr @ we_a

    al = s_src[src] + s_dst[dst] + eal
    al = jnp.where(al > 0, al, _NEG * al)
    shift = jnp.maximum(s_dst, 0.0)
    ex = jnp.exp(al - shift[dst])
    den = jax.ops.segment_sum(ex, dst, num_segments=n)
    alpha = ex / (den[dst] + 1e-16)
    out = jax.ops.segment_sum(
        h[src].reshape(-1, H, C) * alpha[..., None], dst,
        num_segments=n).reshape(n, H * C)
    return out + b


def _ln_relu(x, g, b):
    mu = x.mean(-1, keepdims=True)
    var = ((x - mu) ** 2).mean(-1, keepdims=True)
    return jax.nn.relu((x - mu) / jnp.sqrt(var + 1e-5) * g + b)


def kernel(x, edge_index, edge_attr, W1, We1, as1, ad1, ae1, b1, g1, bb1,
           W2, We2, as2, ad2, ae2, b2, g2, bb2):
    src = edge_index[0]
    dst = edge_index[1]
    h = _gat_layer(x, src, dst, edge_attr, W1, We1, as1, ad1, ae1, b1, 4, 512)
    h = _ln_relu(h, g1, bb1)
    h = _gat_layer(h, src, dst, edge_attr, W2, We2, as2, ad2, ae2, b2, 3, 481)
    return _ln_relu(h, g2, bb2)


# speedup vs baseline: 5.6569x; 5.6569x over previous
import functools
import jax
import jax.numpy as jnp
from jax import lax
from jax.experimental import pallas as pl
from jax.experimental.pallas import tpu as pltpu
from jax.experimental.pallas import tpu_sc as plsc

_NEG = 0.2
_NC, _NS, _L = 2, 16, 16
_B = 80
_CW = 128


def _sc_aggregate(hflat, src3d, dst3d, ex4d, zrows, *, F, N, E, H):
    NP = 10240
    NPC = NP // _NC
    ACC = NPC + 8
    EP = E // _NS
    NB = EP // _B
    NBUF = 2
    NR = NB // NBUF
    FH = F // H
    NT = NPC // _NS
    NG = _B // _L
    mesh = plsc.VectorSubcoreMesh(core_axis_name="c", subcore_axis_name="s")

    @functools.partial(
        pl.kernel, mesh=mesh,
        out_type=jax.ShapeDtypeStruct((F * NP, _CW), jnp.float32),
        scratch_types=[
            pltpu.VMEM((1, EP), jnp.int32),
            pltpu.VMEM((1, EP), jnp.int32),
            pltpu.VMEM((1, EP), jnp.float32),
            pltpu.VMEM((NBUF, _B, _CW), jnp.float32),
            pltpu.VMEM((NBUF, _B), jnp.int32),
            pltpu.VMEM((NBUF, _B), jnp.int32),
            pltpu.VMEM((NBUF, _B), jnp.float32),
            pltpu.VMEM_SHARED((ACC, _CW), jnp.float32),
            [pltpu.SemaphoreType.DMA] * NBUF,
            [pltpu.SemaphoreType.DMA] * NBUF,
        ],
    )
    def agg(hflat_h, src_h, dst_h, ex_h, z_h, out_h,
            srcv, dstv, wv, rows, idxb, dstb, wb, acc, gsem, ssem):
        c = lax.axis_index("c")
        s = lax.axis_index("s")
        base = c * NPC
        pltpu.sync_copy(src_h.at[s], srcv)
        pltpu.sync_copy(dst_h.at[s], dstv)

        def prep(b, bi, hbase):
            for g in range(NG):
                sl = pl.ds(g * _L, _L)
                esl = pl.ds(bi * _B + g * _L, _L)
                sv = srcv[0, esl]
                dv = dstv[0, esl]
                wvv = wv[0, esl]
                m = (dv >= base) & (dv < base + NPC)
                idxb[b, sl] = sv + hbase
                dstb[b, sl] = jnp.where(m, dv - base, NPC)
                wb[b, sl] = jnp.where(m, wvv, 0.0)

        def gather_start(b):
            pltpu.async_copy(hflat_h.at[idxb.at[b]], rows.at[b], gsem[b])

        def gather_wait(b):
            pltpu.make_async_copy(hflat_h.at[idxb.at[b]], rows.at[b],
                                  gsem[b]).wait()

        def scatter_start(b):
            pltpu.async_copy(rows.at[b], acc.at[dstb.at[b]], ssem[b],
                             add=True)

        def scatter_wait(b):
            pltpu.make_async_copy(rows.at[b], acc.at[dstb.at[b]],
                                  ssem[b]).wait()

        def scale(b):
            def sgroup(gi, _):
                wg2 = wb[b, pl.ds(gi * _L, _L)]
                for t in range(_L):
                    w = jnp.full((_L,), wg2[t], jnp.float32)
                    for gg in range(_CW // _L):
                        sl2 = pl.ds(gg * _L, _L)
                        rows[b, gi * _L + t, sl2] = (
                            rows[b, gi * _L + t, sl2] * w)
                return 0

            lax.fori_loop(0, NG, sgroup, 0, unroll=False)

        def chunk(f, _):
            hf = f // FH
            hbase = f * N
            pltpu.sync_copy(ex_h.at[hf, s], wv)
            pltpu.sync_copy(z_h, acc.at[pl.ds(s * NT, NT)])

            @pl.when(s == 0)
            def _():
                pltpu.sync_copy(z_h.at[pl.ds(0, 8)], acc.at[pl.ds(NPC, 8)])

            plsc.subcore_barrier()

            for b in range(NBUF):
                prep(b, jnp.int32(b), hbase)
                gather_start(b)

            def rnd(r, _):
                for b in range(NBUF):
                    bi = r * NBUF + b
                    gather_wait(b)
                    scale(b)
                    scatter_start(b)
                    if b >= 1:
                        scatter_wait(b - 1)
                        prep(b - 1, jnp.minimum(bi - 1 + NBUF, NB - 1), hbase)
                        gather_start(b - 1)
                scatter_wait(NBUF - 1)
                prep(NBUF - 1,
                     jnp.minimum(r * NBUF + 2 * NBUF - 1, NB - 1), hbase)
                gather_start(NBUF - 1)
                return 0

            lax.fori_loop(0, NR, rnd, 0, unroll=False)
            for b in range(NBUF):
                gather_wait(b)
            plsc.subcore_barrier()
            pltpu.sync_copy(acc.at[pl.ds(s * NT, NT)],
                            out_h.at[pl.ds(f * NP + base + s * NT, NT)])
            plsc.subcore_barrier()
            return 0

        lax.fori_loop(0, F, chunk, 0, unroll=False)

    return agg(hflat, src3d, dst3d, ex4d, zrows)


def _matmul_kernel(x_ref, w_ref, o_ref, acc_ref, *, nsteps):
    k = pl.program_id(2)

    @pl.when(k == 0)
    def _():
        acc_ref[...] = jnp.zeros_like(acc_ref)

    acc_ref[...] += jnp.dot(x_ref[...], w_ref[...],
                            preferred_element_type=jnp.float32)

    @pl.when(k == nsteps - 1)
    def _():
        o_ref[...] = acc_ref[...]


def _matmul(x, w, bm=256, bn=512, bk=128):
    m, k = x.shape
    k2, n = w.shape
    assert k == k2
    nsteps = pl.cdiv(k, bk)
    return pl.pallas_call(
        functools.partial(_matmul_kernel, nsteps=nsteps),
        grid=(pl.cdiv(m, bm), pl.cdiv(n, bn), nsteps),
        in_specs=[
            pl.BlockSpec((bm, bk), lambda i, j, kk: (i, kk)),
            pl.BlockSpec((bk, bn), lambda i, j, kk: (kk, j)),
        ],
        out_specs=pl.BlockSpec((bm, bn), lambda i, j, kk: (i, j)),
        scratch_shapes=[pltpu.VMEM((bm, bn), jnp.float32)],
        out_shape=jax.ShapeDtypeStruct((m, n), jnp.float32),
        compiler_params=pltpu.CompilerParams(
            dimension_semantics=("parallel", "parallel", "arbitrary")),
    )(x, w)


def _gat_layer(x, src, dst, src2d, dst2d, edge_attr,
               W, We, a_src, a_dst, a_e, b, H, C):
    n = x.shape[0]
    d = x.shape[1]
    e = src.shape[0]
    Cp = 512
    F = H * Cp // _CW
    if C != Cp:
        Wp = jnp.zeros((d, H * Cp), jnp.float32)
        for hh_ in range(H):
            Wp = lax.dynamic_update_slice(
                Wp, W[:, hh_ * C:(hh_ + 1) * C], (0, hh_ * Cp))
    else:
        Wp = W
    w_s = (W.reshape(d, H, C) * a_src[None]).sum(-1)
    w_d = (W.reshape(d, H, C) * a_dst[None]).sum(-1)
    pad = 128 - 2 * H
    w_aug = jnp.concatenate(
        [Wp, w_s, w_d, jnp.zeros((d, pad), jnp.float32)], axis=1)
    hh = _matmul(x, w_aug)
    s_src = hh[:, H * Cp:H * Cp + H]
    s_dst = hh[:, H * Cp + H:H * Cp + 2 * H]

    we_a = (We.reshape(-1, H, C) * a_e[None]).sum(-1)
    eal = edge_attr @ we_a

    al = s_src[src] + s_dst[dst] + eal
    al = jnp.where(al > 0, al, _NEG * al)
    shift = jnp.maximum(s_dst, 0.0)
    ex = jnp.exp(al - shift[dst])
    den = jax.ops.segment_sum(ex, dst, num_segments=n)

    hflat = hh[:, :H * Cp].reshape(n, F, _CW).transpose(1, 0, 2)
    hflat = hflat.reshape(F * n, _CW)
    ex4d = ex.T.reshape(H, _NS, 1, e // _NS)
    NP = 10240
    zrows = jnp.zeros((NP // _NC // _NS, _CW), jnp.float32)
    outflat = _sc_aggregate(hflat, src2d, dst2d, ex4d, zrows,
                            F=F, N=n, E=e, H=H)
    out = outflat.reshape(F, NP, _CW)[:, :n]
    out = out.transpose(1, 0, 2).reshape(n, H * Cp)
    out = out.reshape(n, H, Cp)[:, :, :C]
    out = out / (den[..., None] + 1e-16)
    return out.reshape(n, H * C) + b


def _ln_relu(x, g, b):
    mu = x.mean(-1, keepdims=True)
    var = ((x - mu) ** 2).mean(-1, keepdims=True)
    return jax.nn.relu((x - mu) / jnp.sqrt(var + 1e-5) * g + b)


def kernel(x, edge_index, edge_attr, W1, We1, as1, ad1, ae1, b1, g1, bb1,
           W2, We2, as2, ad2, ae2, b2, g2, bb2):
    src = edge_index[0]
    dst = edge_index[1]
    e = src.shape[0]
    src2d = src.reshape(_NS, 1, e // _NS)
    dst2d = dst.reshape(_NS, 1, e // _NS)
    h = _gat_layer(x, src, dst, src2d, dst2d, edge_attr,
                   W1, We1, as1, ad1, ae1, b1, 4, 512)
    h = _ln_relu(h, g1, bb1)
    h = _gat_layer(h, src, dst, src2d, dst2d, edge_attr,
                   W2, We2, as2, ad2, ae2, b2, 3, 481)
    return _ln_relu(h, g2, bb2)
